# baseline (speedup 1.0000x reference)
import functools

import jax
import jax.numpy as jnp
from jax import lax
from jax.experimental import pallas as pl
from jax.experimental.pallas import tpu as pltpu
from jax.experimental.pallas import tpu_sc as plsc

N = 10000
E = 160000
P = 20000
D1 = 256
DH = 512

NC = 2
NS = 16
H = N // NC
HP = 5120
STRIPE = HP // NS
EP = E // NS
CHUNK = 128
NCH = 128
CAP = NCH * CHUNK
ACC_R = STRIPE + 8

PC = 640


def _sc_mesh():
    return plsc.VectorSubcoreMesh(core_axis_name="c", subcore_axis_name="s",
                                  num_cores=NC)


def _prep(edge_index):
    @functools.partial(
        pl.kernel,
        mesh=_sc_mesh(),
        compiler_params=pltpu.CompilerParams(needs_layout_passes=False),
        out_type=[
            jax.ShapeDtypeStruct((NC * HP * 16,), jnp.float32),
            jax.ShapeDtypeStruct((NC * NS * CAP,), jnp.int32),
            jax.ShapeDtypeStruct((NC * NS * CAP,), jnp.int32),
            jax.ShapeDtypeStruct((NC * NS * 16,), jnp.int32),
        ],
        scratch_types=[
            pltpu.VMEM((EP,), jnp.int32),
            pltpu.VMEM((EP,), jnp.int32),
            pltpu.VMEM((CAP + 16,), jnp.int32),
            pltpu.VMEM((CAP + 16,), jnp.int32),
            pltpu.VMEM((ACC_R * 16,), jnp.float32),
            pltpu.VMEM((16,), jnp.int32),
        ],
    )
    def k(ei_hbm, deg_hbm, srcl_hbm, dloc_hbm, cnt_hbm,
          src_v, dst_v, csrc_v, cdl_v, dacc, cnt_v):
        c = lax.axis_index("c")
        s = lax.axis_index("s")
        wid = c * NS + s
        low = c * H + s * STRIPE
        high = jnp.minimum(low + STRIPE, c * H + H)

        def fill(i, _):
            csrc_v[pl.ds(i * 16, 16)] = jnp.zeros((16,), jnp.int32)
            cdl_v[pl.ds(i * 16, 16)] = jnp.full((16,), STRIPE, jnp.int32)
            return 0
        lax.fori_loop(0, (CAP + 16) // 16, fill, 0)

        def zdeg(i, _):
            dacc[pl.ds(i * 16, 16)] = jnp.zeros((16,), jnp.float32)
            return 0
        lax.fori_loop(0, ACC_R, zdeg, 0)

        def slab(t, n):
            pltpu.sync_copy(ei_hbm.at[pl.ds(t * EP, EP)], src_v)
            pltpu.sync_copy(ei_hbm.at[pl.ds(E + t * EP, EP)], dst_v)

            def compact(g, n):
                sv = src_v[pl.ds(g * 16, 16)]
                dv = dst_v[pl.ds(g * 16, 16)]
                m = jnp.logical_and(dv >= low, dv < high)
                mi = m.astype(jnp.int32)
                inc = plsc.cumsum(mi)
                pos = jnp.where(m, n + inc - mi, CAP)
                pos = jnp.minimum(pos, CAP)
                plsc.store_scatter(csrc_v, [pos], sv)
                plsc.store_scatter(cdl_v, [pos], dv - low)
                return n + plsc.all_reduce_population_count(m)[0]
            return lax.fori_loop(0, EP // 16, compact, n)
        n = lax.fori_loop(0, NS, slab, jnp.int32(0))
        nch = jnp.minimum((n + CHUNK - 1) // CHUNK, NCH)

        ones = jnp.ones((16,), jnp.float32)

        def dcount(g, _):
            dlv = cdl_v[pl.ds(g * 16, 16)]
            for k_ in range(16):
                o = dlv[k_]
                plsc.addupdate(dacc.at[pl.ds(o * 16, 16)], ones)
            return 0
        lax.fori_loop(0, nch * (CHUNK // 16), dcount, 0)

        pltpu.sync_copy(dacc.at[pl.ds(0, STRIPE * 16)],
                        deg_hbm.at[pl.ds((c * HP + s * STRIPE) * 16, STRIPE * 16)])
        pltpu.sync_copy(csrc_v.at[pl.ds(0, CAP)], srcl_hbm.at[pl.ds(wid * CAP, CAP)])
        pltpu.sync_copy(cdl_v.at[pl.ds(0, CAP)], dloc_hbm.at[pl.ds(wid * CAP, CAP)])
        cnt_v[pl.ds(0, 16)] = jnp.full((16,), 1, jnp.int32) * nch
        pltpu.sync_copy(cnt_v, cnt_hbm.at[pl.ds(wid * 16, 16)])

    return k(edge_index)


def _agg(featp, srcl, dloc, cnt, zflat):
    @functools.partial(
        pl.kernel,
        mesh=_sc_mesh(),
        compiler_params=pltpu.CompilerParams(needs_layout_passes=False),
        out_type=jax.ShapeDtypeStruct((NC * HP * D1,), jnp.float32),
        scratch_types=[
            pltpu.VMEM((CHUNK,), jnp.int32),
            pltpu.VMEM((CHUNK,), jnp.int32),
            pltpu.VMEM((16,), jnp.int32),
            pltpu.VMEM((CHUNK, D1), jnp.float32),
            pltpu.VMEM((ACC_R * D1,), jnp.float32),
            pltpu.SemaphoreType.DMA,
        ],
    )
    def k(feat_hbm, srcl_hbm, dloc_hbm, cnt_hbm, z_hbm, out_hbm,
          idx_v, dl_v, cnt_v, buf, acc, sem):
        c = lax.axis_index("c")
        s = lax.axis_index("s")
        wid = c * NS + s

        pltpu.sync_copy(z_hbm.at[pl.ds(0, ACC_R * D1)], acc)
        pltpu.sync_copy(cnt_hbm.at[pl.ds(wid * 16, 16)], cnt_v)
        nch = cnt_v[pl.ds(0, 16)][0]

        def body(j, _):
            pltpu.sync_copy(srcl_hbm.at[pl.ds(wid * CAP + j * CHUNK, CHUNK)], idx_v)
            pltpu.sync_copy(dloc_hbm.at[pl.ds(wid * CAP + j * CHUNK, CHUNK)], dl_v)
            pltpu.async_copy(feat_hbm.at[idx_v], buf, sem).wait()

            def grp(g, _):
                dlv = dl_v[pl.ds(g * 16, 16)]
                adst = dlv * D1
                erow = lax.iota(jnp.int32, 16) + g * 16

                def col(ci, _):
                    cvec = jnp.full((16,), 1, jnp.int32) * ci
                    vals = plsc.load_gather(buf, [erow, cvec])
                    plsc.addupdate_scatter(acc, [adst + ci], vals)
                    return 0
                lax.fori_loop(0, D1, col, 0)
                return 0
            lax.fori_loop(0, CHUNK // 16, grp, 0)
            return 0
        lax.fori_loop(0, nch, body, 0)

        pltpu.sync_copy(acc.at[pl.ds(0, STRIPE * D1)],
                        out_hbm.at[pl.ds((c * HP + s * STRIPE) * D1, STRIPE * D1)])

    return k(featp, srcl, dloc, cnt, zflat)


def _decode(s_arr, t_arr, pei):
    @functools.partial(
        pl.kernel,
        mesh=_sc_mesh(),
        compiler_params=pltpu.CompilerParams(needs_layout_passes=False),
        out_type=jax.ShapeDtypeStruct((P,), jnp.float32),
        scratch_types=[
            pltpu.VMEM((N,), jnp.float32),
            pltpu.VMEM((N,), jnp.float32),
            pltpu.VMEM((PC,), jnp.int32),
            pltpu.VMEM((PC,), jnp.int32),
            pltpu.VMEM((PC,), jnp.float32),
        ],
    )
    def k(s_hbm, t_hbm, pei_hbm, out_hbm, sv, tv, rv, cv, ov):
        c = lax.axis_index("c")
        s = lax.axis_index("s")
        wid = c * NS + s
        off = jnp.minimum(wid * PC, P - PC)

        pltpu.sync_copy(s_hbm, sv)
        pltpu.sync_copy(t_hbm, tv)
        pltpu.sync_copy(pei_hbm.at[pl.ds(off, PC)], rv)
        pltpu.sync_copy(pei_hbm.at[pl.ds(P + off, PC)], cv)

        def body(g, _):
            ir = rv[pl.ds(g * 16, 16)]
            ic = cv[pl.ds(g * 16, 16)]
            vs = plsc.load_gather(sv, [ir])
            vt = plsc.load_gather(tv, [ic])
            u = vs + vt
            ov[pl.ds(g * 16, 16)] = 1.0 / (1.0 + jnp.exp(-u))
            return 0
        lax.fori_loop(0, PC // 16, body, 0)

        pltpu.sync_copy(ov, out_hbm.at[pl.ds(off, PC)])

    return k(s_arr, t_arr, pei)


_BLK = 1000


def _tc_scale(deg, x):
    def body(deg_ref, x_ref, dinv_ref, fp_ref):
        dv = lax.rsqrt(deg_ref[...] + 1.0)
        dinv_ref[...] = dv
        fp_ref[...] = x_ref[...] * dv

    return pl.pallas_call(
        body,
        grid=(N // _BLK,),
        in_specs=[pl.BlockSpec((_BLK, 1), lambda i: (i, 0)),
                  pl.BlockSpec((_BLK, D1), lambda i: (i, 0))],
        out_specs=[pl.BlockSpec((_BLK, 1), lambda i: (i, 0)),
                   pl.BlockSpec((_BLK, D1), lambda i: (i, 0))],
        out_shape=[jax.ShapeDtypeStruct((N, 1), jnp.float32),
                   jax.ShapeDtypeStruct((N, D1), jnp.float32)],
    )(deg, x)


def _tc_mlp(dinv, agg1, featp1, W1, b1, W2):
    def body(dinv_ref, agg_ref, fp_ref, W1_ref, b1_ref, W2_ref, out_ref):
        a = dinv_ref[...] * (agg_ref[...] + fp_ref[...])
        h = jnp.dot(a, W1_ref[...], preferred_element_type=jnp.float32)
        h = jnp.maximum(h + b1_ref[...], 0.0)
        hw = jnp.dot(h, W2_ref[...], preferred_element_type=jnp.float32)
        out_ref[...] = dinv_ref[...] * hw

    return pl.pallas_call(
        body,
        grid=(N // _BLK,),
        in_specs=[pl.BlockSpec((_BLK, 1), lambda i: (i, 0)),
                  pl.BlockSpec((_BLK, D1), lambda i: (i, 0)),
                  pl.BlockSpec((_BLK, D1), lambda i: (i, 0)),
                  pl.BlockSpec((D1, DH), lambda i: (0, 0)),
                  pl.BlockSpec((1, DH), lambda i: (0, 0)),
                  pl.BlockSpec((DH, D1), lambda i: (0, 0))],
        out_specs=pl.BlockSpec((_BLK, D1), lambda i: (i, 0)),
        out_shape=jax.ShapeDtypeStruct((N, D1), jnp.float32),
    )(dinv, agg1, featp1, W1, b1, W2)


def _tc_final(dinv, agg2, featp2, b2, Wcat, bvec):
    def body(dinv_ref, agg_ref, fp_ref, b2_ref, Wc_ref, bv_ref, out_ref):
        z = dinv_ref[...] * (agg_ref[...] + fp_ref[...]) + b2_ref[...]
        out_ref[...] = jnp.dot(z, Wc_ref[...],
                               preferred_element_type=jnp.float32) + bv_ref[...]

    return pl.pallas_call(
        body,
        grid=(N // _BLK,),
        in_specs=[pl.BlockSpec((_BLK, 1), lambda i: (i, 0)),
                  pl.BlockSpec((_BLK, D1), lambda i: (i, 0)),
                  pl.BlockSpec((_BLK, D1), lambda i: (i, 0)),
                  pl.BlockSpec((1, D1), lambda i: (0, 0)),
                  pl.BlockSpec((D1, 128), lambda i: (0, 0)),
                  pl.BlockSpec((1, 128), lambda i: (0, 0))],
        out_specs=pl.BlockSpec((_BLK, 128), lambda i: (i, 0)),
        out_shape=jax.ShapeDtypeStruct((N, 128), jnp.float32),
    )(dinv, agg2, featp2, b2, Wcat, bvec)


def kernel(x, edge_index, predict_edge_index, W1, b1, W2, b2, Wout, bout):
    zflat = jnp.zeros((ACC_R * D1,), jnp.float32)

    deg16, srcl, dloc, cnt = _prep(edge_index.reshape(2 * E))
    dd = deg16.reshape(NC * HP, 16)[:, 0]
    deg = jnp.concatenate([dd[:H], dd[HP:HP + H]]).reshape(N, 1)

    dinv, featp1 = _tc_scale(deg, x)

    a1 = _agg(featp1, srcl, dloc, cnt, zflat).reshape(NC * HP, D1)
    agg1 = jnp.concatenate([a1[:H], a1[HP:HP + H]])
    featp2 = _tc_mlp(dinv, agg1, featp1, W1, b1.reshape(1, DH), W2)
    a2 = _agg(featp2, srcl, dloc, cnt, zflat).reshape(NC * HP, D1)
    agg2 = jnp.concatenate([a2[:H], a2[HP:HP + H]])

    Wcat = jnp.concatenate([Wout[:D1], Wout[D1:]], axis=1)
    Wcat = jnp.pad(Wcat, ((0, 0), (0, 126)))
    bvec = jnp.concatenate([bout, jnp.zeros((127,), jnp.float32)]).reshape(1, 128)
    st = _tc_final(dinv, agg2, featp2, b2.reshape(1, D1), Wcat, bvec)

    s_arr = st[:, 0]
    t_arr = st[:, 1]
    return _decode(s_arr, t_arr, predict_edge_index.reshape(2 * P))


# speedup vs baseline: 3.7389x; 3.7389x over previous
import functools

import jax
import jax.numpy as jnp
from jax import lax
from jax.experimental import pallas as pl
from jax.experimental.pallas import tpu as pltpu
from jax.experimental.pallas import tpu_sc as plsc

N = 10000
E = 160000
P = 20000
D1 = 256
DH = 512

NC = 2
NS = 16
H = N // NC
HP = 5120
STRIPE = HP // NS
EP = E // NS
CHUNK = 64
NCH = 158
CAP = NCH * CHUNK
ACC_R = STRIPE + 1

PC = 640


def _sc_mesh():
    return plsc.VectorSubcoreMesh(core_axis_name="c", subcore_axis_name="s",
                                  num_cores=NC)


def _prep(edge_index):
    @functools.partial(
        pl.kernel,
        mesh=_sc_mesh(),
        compiler_params=pltpu.CompilerParams(needs_layout_passes=False),
        out_type=[
            jax.ShapeDtypeStruct((NC * HP * 16,), jnp.float32),
            jax.ShapeDtypeStruct((NC * NS * 2 * CAP,), jnp.int32),
            jax.ShapeDtypeStruct((NC * NS * 16,), jnp.int32),
        ],
        scratch_types=[
            pltpu.VMEM((EP,), jnp.int32),
            pltpu.VMEM((EP,), jnp.int32),
            pltpu.VMEM((CAP + 16,), jnp.int32),
            pltpu.VMEM((CAP + 16,), jnp.int32),
            pltpu.VMEM((2 * CAP,), jnp.int32),
            pltpu.VMEM((ACC_R * 16,), jnp.float32),
            pltpu.VMEM((16,), jnp.int32),
        ],
    )
    def k(ei_hbm, deg_hbm, lst_hbm, cnt_hbm,
          src_v, dst_v, csrc_v, cdl_v, ilist_v, dacc, cnt_v):
        c = lax.axis_index("c")
        s = lax.axis_index("s")
        wid = c * NS + s
        low = c * H + s * STRIPE
        high = jnp.minimum(low + STRIPE, c * H + H)

        def fill(i, _):
            csrc_v[pl.ds(i * 16, 16)] = jnp.zeros((16,), jnp.int32)
            cdl_v[pl.ds(i * 16, 16)] = jnp.full((16,), STRIPE, jnp.int32)
            return 0
        lax.fori_loop(0, (CAP + 16) // 16, fill, 0)

        def zdeg(i, _):
            dacc[pl.ds(i * 16, 16)] = jnp.zeros((16,), jnp.float32)
            return 0
        lax.fori_loop(0, ACC_R, zdeg, 0)

        def slab(t, n):
            pltpu.sync_copy(ei_hbm.at[pl.ds(t * EP, EP)], src_v)
            pltpu.sync_copy(ei_hbm.at[pl.ds(E + t * EP, EP)], dst_v)

            def compact(g, n):
                sv = src_v[pl.ds(g * 16, 16)]
                dv = dst_v[pl.ds(g * 16, 16)]
                m = jnp.logical_and(dv >= low, dv < high)
                mi = m.astype(jnp.int32)
                inc = plsc.cumsum(mi)
                pos = jnp.where(m, n + inc - mi, CAP)
                pos = jnp.minimum(pos, CAP)
                plsc.store_scatter(csrc_v, [pos], sv)
                plsc.store_scatter(cdl_v, [pos], dv - low)
                return n + plsc.all_reduce_population_count(m)[0]
            return lax.fori_loop(0, EP // 16, compact, n)
        n = lax.fori_loop(0, NS, slab, jnp.int32(0))
        nch = jnp.minimum((n + CHUNK - 1) // CHUNK, NCH)

        ones = jnp.ones((16,), jnp.float32)

        def dcount(g, _):
            dlv = cdl_v[pl.ds(g * 16, 16)]
            for k_ in range(16):
                o = dlv[k_]
                plsc.addupdate(dacc.at[pl.ds(o * 16, 16)], ones)
            return 0
        lax.fori_loop(0, nch * (CHUNK // 16), dcount, 0)

        def inter(j, _):
            def cpg(g, _):
                ilist_v[pl.ds(j * 2 * CHUNK + g * 16, 16)] = \
                    csrc_v[pl.ds(j * CHUNK + g * 16, 16)]
                ilist_v[pl.ds(j * 2 * CHUNK + CHUNK + g * 16, 16)] = \
                    cdl_v[pl.ds(j * CHUNK + g * 16, 16)]
                return 0
            lax.fori_loop(0, CHUNK // 16, cpg, 0)
            return 0
        lax.fori_loop(0, nch, inter, 0)

        pltpu.sync_copy(dacc.at[pl.ds(0, STRIPE * 16)],
                        deg_hbm.at[pl.ds((c * HP + s * STRIPE) * 16, STRIPE * 16)])
        pltpu.sync_copy(ilist_v, lst_hbm.at[pl.ds(wid * 2 * CAP, 2 * CAP)])
        cnt_v[pl.ds(0, 16)] = jnp.full((16,), 1, jnp.int32) * nch
        pltpu.sync_copy(cnt_v, cnt_hbm.at[pl.ds(wid * 16, 16)])

    return k(edge_index)


def _agg(featp, lst, cnt, zflat):
    @functools.partial(
        pl.kernel,
        mesh=_sc_mesh(),
        compiler_params=pltpu.CompilerParams(needs_layout_passes=False),
        out_type=jax.ShapeDtypeStruct((NC * HP * D1,), jnp.float32),
        scratch_types=[
            pltpu.VMEM((2 * CHUNK,), jnp.int32),
            pltpu.VMEM((2 * CHUNK,), jnp.int32),
            pltpu.VMEM((16,), jnp.int32),
            pltpu.VMEM((CHUNK, D1), jnp.float32),
            pltpu.VMEM((CHUNK, D1), jnp.float32),
            pltpu.VMEM((ACC_R * D1,), jnp.float32),
            pltpu.SemaphoreType.DMA,
            pltpu.SemaphoreType.DMA,
            pltpu.SemaphoreType.DMA,
            pltpu.SemaphoreType.DMA,
        ],
    )
    def k(feat_hbm, lst_hbm, cnt_hbm, z_hbm, out_hbm,
          ilA, ilB, cnt_v, bufA, bufB, acc, isA, isB, gsA, gsB):
        c = lax.axis_index("c")
        s = lax.axis_index("s")
        wid = c * NS + s
        lbase = wid * 2 * CAP

        pltpu.sync_copy(z_hbm.at[pl.ds(0, ACC_R * D1)], acc)
        pltpu.sync_copy(cnt_hbm.at[pl.ds(wid * 16, 16)], cnt_v)
        nch = cnt_v[pl.ds(0, 16)][0]

        def stage(j, il, sem):
            pltpu.async_copy(lst_hbm.at[pl.ds(lbase + j * 2 * CHUNK, 2 * CHUNK)],
                             il, sem)

        def gather(il, buf, gsem):
            pltpu.async_copy(feat_hbm.at[il.at[pl.ds(0, CHUNK)]], buf, gsem)

        def wait_list(il, sem):
            pltpu.make_async_copy(lst_hbm.at[pl.ds(0, 2 * CHUNK)], il, sem).wait()

        def wait_rows(buf, gsem):
            pltpu.make_async_copy(feat_hbm.at[pl.ds(0, CHUNK)], buf, gsem).wait()

        def accumulate(il, buf):
            def grp(g, _):
                dlv = il[pl.ds(CHUNK + g * 16, 16)]
                for k_ in range(16):
                    o = dlv[k_]
                    e = g * 16 + k_
                    for i_ in range(D1 // 16):
                        plsc.addupdate(
                            acc.at[pl.ds(o * D1 + i_ * 16, 16)],
                            buf[e, pl.ds(i_ * 16, 16)])
                return 0
            lax.fori_loop(0, CHUNK // 16, grp, 0)

        @pl.when(nch > 0)
        def _():
            stage(0, ilA, isA)
            wait_list(ilA, isA)
            gather(ilA, bufA, gsA)

        @pl.when(nch > 1)
        def _():
            stage(1, ilB, isB)

        npairs = (nch + 1) // 2

        def pair(p, _):
            j0 = p * 2
            j1 = j0 + 1
            @pl.when(j1 < nch)
            def _():
                wait_list(ilB, isB)
                gather(ilB, bufB, gsB)
            wait_rows(bufA, gsA)
            accumulate(ilA, bufA)
            @pl.when(j0 + 2 < nch)
            def _():
                stage(j0 + 2, ilA, isA)
            @pl.when(j1 < nch)
            def _():
                @pl.when(j1 + 1 < nch)
                def _():
                    wait_list(ilA, isA)
                    gather(ilA, bufA, gsA)
                wait_rows(bufB, gsB)
                accumulate(ilB, bufB)
                @pl.when(j1 + 2 < nch)
                def _():
                    stage(j1 + 2, ilB, isB)
            return 0
        lax.fori_loop(0, npairs, pair, 0)

        pltpu.sync_copy(acc.at[pl.ds(0, STRIPE * D1)],
                        out_hbm.at[pl.ds((c * HP + s * STRIPE) * D1, STRIPE * D1)])

    return k(featp, lst, cnt, zflat)


def _decode(s_arr, t_arr, pei):
    @functools.partial(
        pl.kernel,
        mesh=_sc_mesh(),
        compiler_params=pltpu.CompilerParams(needs_layout_passes=False),
        out_type=jax.ShapeDtypeStruct((P,), jnp.float32),
        scratch_types=[
            pltpu.VMEM((N,), jnp.float32),
            pltpu.VMEM((N,), jnp.float32),
            pltpu.VMEM((PC,), jnp.int32),
            pltpu.VMEM((PC,), jnp.int32),
            pltpu.VMEM((PC,), jnp.float32),
        ],
    )
    def k(s_hbm, t_hbm, pei_hbm, out_hbm, sv, tv, rv, cv, ov):
        c = lax.axis_index("c")
        s = lax.axis_index("s")
        wid = c * NS + s
        off = jnp.minimum(wid * PC, P - PC)

        pltpu.sync_copy(s_hbm, sv)
        pltpu.sync_copy(t_hbm, tv)
        pltpu.sync_copy(pei_hbm.at[pl.ds(off, PC)], rv)
        pltpu.sync_copy(pei_hbm.at[pl.ds(P + off, PC)], cv)

        def body(g, _):
            ir = rv[pl.ds(g * 16, 16)]
            ic = cv[pl.ds(g * 16, 16)]
            vs = plsc.load_gather(sv, [ir])
            vt = plsc.load_gather(tv, [ic])
            u = vs + vt
            ov[pl.ds(g * 16, 16)] = 1.0 / (1.0 + jnp.exp(-u))
            return 0
        lax.fori_loop(0, PC // 16, body, 0)

        pltpu.sync_copy(ov, out_hbm.at[pl.ds(off, PC)])

    return k(s_arr, t_arr, pei)


_BLK = 1000


def _tc_scale(deg, x):
    def body(deg_ref, x_ref, dinv_ref, fp_ref):
        dv = lax.rsqrt(deg_ref[...] + 1.0)
        dinv_ref[...] = dv
        fp_ref[...] = x_ref[...] * dv

    return pl.pallas_call(
        body,
        grid=(N // _BLK,),
        in_specs=[pl.BlockSpec((_BLK, 1), lambda i: (i, 0)),
                  pl.BlockSpec((_BLK, D1), lambda i: (i, 0))],
        out_specs=[pl.BlockSpec((_BLK, 1), lambda i: (i, 0)),
                   pl.BlockSpec((_BLK, D1), lambda i: (i, 0))],
        out_shape=[jax.ShapeDtypeStruct((N, 1), jnp.float32),
                   jax.ShapeDtypeStruct((N, D1), jnp.float32)],
    )(deg, x)


def _tc_mlp(dinv, agg1, featp1, W1, b1, W2):
    def body(dinv_ref, agg_ref, fp_ref, W1_ref, b1_ref, W2_ref, out_ref):
        a = dinv_ref[...] * (agg_ref[...] + fp_ref[...])
        h = jnp.dot(a, W1_ref[...], preferred_element_type=jnp.float32)
        h = jnp.maximum(h + b1_ref[...], 0.0)
        hw = jnp.dot(h, W2_ref[...], preferred_element_type=jnp.float32)
        out_ref[...] = dinv_ref[...] * hw

    return pl.pallas_call(
        body,
        grid=(N // _BLK,),
        in_specs=[pl.BlockSpec((_BLK, 1), lambda i: (i, 0)),
                  pl.BlockSpec((_BLK, D1), lambda i: (i, 0)),
                  pl.BlockSpec((_BLK, D1), lambda i: (i, 0)),
                  pl.BlockSpec((D1, DH), lambda i: (0, 0)),
                  pl.BlockSpec((1, DH), lambda i: (0, 0)),
                  pl.BlockSpec((DH, D1), lambda i: (0, 0))],
        out_specs=pl.BlockSpec((_BLK, D1), lambda i: (i, 0)),
        out_shape=jax.ShapeDtypeStruct((N, D1), jnp.float32),
    )(dinv, agg1, featp1, W1, b1, W2)


def _tc_final(dinv, agg2, featp2, b2, Wcat, bvec):
    def body(dinv_ref, agg_ref, fp_ref, b2_ref, Wc_ref, bv_ref, out_ref):
        z = dinv_ref[...] * (agg_ref[...] + fp_ref[...]) + b2_ref[...]
        out_ref[...] = jnp.dot(z, Wc_ref[...],
                               preferred_element_type=jnp.float32) + bv_ref[...]

    return pl.pallas_call(
        body,
        grid=(N // _BLK,),
        in_specs=[pl.BlockSpec((_BLK, 1), lambda i: (i, 0)),
                  pl.BlockSpec((_BLK, D1), lambda i: (i, 0)),
                  pl.BlockSpec((_BLK, D1), lambda i: (i, 0)),
                  pl.BlockSpec((1, D1), lambda i: (0, 0)),
                  pl.BlockSpec((D1, 128), lambda i: (0, 0)),
                  pl.BlockSpec((1, 128), lambda i: (0, 0))],
        out_specs=pl.BlockSpec((_BLK, 128), lambda i: (i, 0)),
        out_shape=jax.ShapeDtypeStruct((N, 128), jnp.float32),
    )(dinv, agg2, featp2, b2, Wcat, bvec)


def kernel(x, edge_index, predict_edge_index, W1, b1, W2, b2, Wout, bout):
    zflat = jnp.zeros((ACC_R * D1,), jnp.float32)

    deg16, lst, cnt = _prep(edge_index.reshape(2 * E))
    dd = deg16.reshape(NC * HP, 16)[:, 0]
    deg = jnp.concatenate([dd[:H], dd[HP:HP + H]]).reshape(N, 1)

    dinv, featp1 = _tc_scale(deg, x)

    a1 = _agg(featp1, lst, cnt, zflat).reshape(NC * HP, D1)
    agg1 = jnp.concatenate([a1[:H], a1[HP:HP + H]])
    featp2 = _tc_mlp(dinv, agg1, featp1, W1, b1.reshape(1, DH), W2)
    a2 = _agg(featp2, lst, cnt, zflat).reshape(NC * HP, D1)
    agg2 = jnp.concatenate([a2[:H], a2[HP:HP + H]])

    Wcat = jnp.concatenate([Wout[:D1], Wout[D1:]], axis=1)
    Wcat = jnp.pad(Wcat, ((0, 0), (0, 126)))
    bvec = jnp.concatenate([bout, jnp.zeros((127,), jnp.float32)]).reshape(1, 128)
    st = _tc_final(dinv, agg2, featp2, b2.reshape(1, D1), Wcat, bvec)

    s_arr = st[:, 0]
    t_arr = st[:, 1]
    return _decode(s_arr, t_arr, predict_edge_index.reshape(2 * P))


# speedup vs baseline: 3.8486x; 1.0294x over previous
import functools

import jax
import jax.numpy as jnp
from jax import lax
from jax.experimental import pallas as pl
from jax.experimental.pallas import tpu as pltpu
from jax.experimental.pallas import tpu_sc as plsc

N = 10000
E = 160000
P = 20000
D1 = 256
DH = 512

NC = 2
NS = 16
H = N // NC
HP = 5120
STRIPE = HP // NS
EP = E // NS
CHUNK = 64
NCH = 158
CAP = NCH * CHUNK
ACC_R = STRIPE + 1

PC = 640


def _sc_mesh():
    return plsc.VectorSubcoreMesh(core_axis_name="c", subcore_axis_name="s",
                                  num_cores=NC)


def _prep(edge_index):
    @functools.partial(
        pl.kernel,
        mesh=_sc_mesh(),
        compiler_params=pltpu.CompilerParams(needs_layout_passes=False),
        out_type=[
            jax.ShapeDtypeStruct((NC * HP * 16,), jnp.float32),
            jax.ShapeDtypeStruct((NC * NS * 2 * CAP,), jnp.int32),
            jax.ShapeDtypeStruct((NC * NS * 16,), jnp.int32),
        ],
        scratch_types=[
            pltpu.VMEM((EP,), jnp.int32),
            pltpu.VMEM((EP,), jnp.int32),
            pltpu.VMEM((EP,), jnp.int32),
            pltpu.VMEM((EP,), jnp.int32),
            pltpu.SemaphoreType.DMA,
            pltpu.SemaphoreType.DMA,
            pltpu.VMEM((CAP + 16,), jnp.int32),
            pltpu.VMEM((CAP + 16,), jnp.int32),
            pltpu.VMEM((2 * CAP,), jnp.int32),
            pltpu.VMEM((ACC_R * 16,), jnp.float32),
            pltpu.VMEM((16,), jnp.int32),
        ],
    )
    def k(ei_hbm, deg_hbm, lst_hbm, cnt_hbm,
          srcA, dstA, srcB, dstB, ssA, ssB, csrc_v, cdl_v, ilist_v, dacc, cnt_v):
        c = lax.axis_index("c")
        s = lax.axis_index("s")
        wid = c * NS + s
        low = c * H + s * STRIPE
        high = jnp.minimum(low + STRIPE, c * H + H)

        def fill(i, _):
            csrc_v[pl.ds(i * 16, 16)] = jnp.zeros((16,), jnp.int32)
            cdl_v[pl.ds(i * 16, 16)] = jnp.full((16,), STRIPE, jnp.int32)
            return 0
        lax.fori_loop(0, (CAP + 16) // 16, fill, 0)

        def zdeg(i, _):
            dacc[pl.ds(i * 16, 16)] = jnp.zeros((16,), jnp.float32)
            return 0
        lax.fori_loop(0, ACC_R, zdeg, 0)

        def sstage(t, sv_ref, dv_ref, sem):
            pltpu.async_copy(ei_hbm.at[pl.ds(t * EP, EP)], sv_ref, sem)
            pltpu.async_copy(ei_hbm.at[pl.ds(E + t * EP, EP)], dv_ref, sem)

        def swait(sv_ref, dv_ref, sem):
            pltpu.make_async_copy(ei_hbm.at[pl.ds(0, EP)], sv_ref, sem).wait()
            pltpu.make_async_copy(ei_hbm.at[pl.ds(0, EP)], dv_ref, sem).wait()

        def comp_slab(src_ref, dst_ref, n):
            def compact(g, n):
                sv = src_ref[pl.ds(g * 16, 16)]
                dv = dst_ref[pl.ds(g * 16, 16)]
                m = jnp.logical_and(dv >= low, dv < high)
                mi = m.astype(jnp.int32)
                inc = plsc.cumsum(mi)
                pos = jnp.where(m, n + inc - mi, CAP)
                pos = jnp.minimum(pos, CAP)
                plsc.store_scatter(csrc_v, [pos], sv)
                plsc.store_scatter(cdl_v, [pos], dv - low)
                return n + plsc.all_reduce_population_count(m)[0]
            return lax.fori_loop(0, EP // 16, compact, n)

        sstage(0, srcA, dstA, ssA)
        sstage(1, srcB, dstB, ssB)

        def spair(p, n):
            t0 = p * 2
            swait(srcA, dstA, ssA)
            n = comp_slab(srcA, dstA, n)

            @pl.when(t0 + 2 < NS)
            def _():
                sstage(t0 + 2, srcA, dstA, ssA)
            swait(srcB, dstB, ssB)
            n = comp_slab(srcB, dstB, n)

            @pl.when(t0 + 3 < NS)
            def _():
                sstage(t0 + 3, srcB, dstB, ssB)
            return n
        n = lax.fori_loop(0, NS // 2, spair, jnp.int32(0))
        nch = jnp.minimum((n + CHUNK - 1) // CHUNK, NCH)

        ones = jnp.ones((16,), jnp.float32)

        def dcount(g, _):
            dlv = cdl_v[pl.ds(g * 16, 16)]
            for k_ in range(16):
                o = dlv[k_]
                plsc.addupdate(dacc.at[pl.ds(o * 16, 16)], ones)
            return 0
        lax.fori_loop(0, nch * (CHUNK // 16), dcount, 0)

        def inter(j, _):
            def cpg(g, _):
                ilist_v[pl.ds(j * 2 * CHUNK + g * 16, 16)] = \
                    csrc_v[pl.ds(j * CHUNK + g * 16, 16)]
                ilist_v[pl.ds(j * 2 * CHUNK + CHUNK + g * 16, 16)] = \
                    cdl_v[pl.ds(j * CHUNK + g * 16, 16)]
                return 0
            lax.fori_loop(0, CHUNK // 16, cpg, 0)
            return 0
        lax.fori_loop(0, nch, inter, 0)

        pltpu.sync_copy(dacc.at[pl.ds(0, STRIPE * 16)],
                        deg_hbm.at[pl.ds((c * HP + s * STRIPE) * 16, STRIPE * 16)])
        pltpu.sync_copy(ilist_v, lst_hbm.at[pl.ds(wid * 2 * CAP, 2 * CAP)])
        cnt_v[pl.ds(0, 16)] = jnp.full((16,), 1, jnp.int32) * nch
        pltpu.sync_copy(cnt_v, cnt_hbm.at[pl.ds(wid * 16, 16)])

    return k(edge_index)


def _agg(featp, lst, cnt, zflat):
    @functools.partial(
        pl.kernel,
        mesh=_sc_mesh(),
        compiler_params=pltpu.CompilerParams(needs_layout_passes=False),
        out_type=jax.ShapeDtypeStruct((NC * HP * D1,), jnp.float32),
        scratch_types=[
            pltpu.VMEM((2 * CHUNK,), jnp.int32),
            pltpu.VMEM((2 * CHUNK,), jnp.int32),
            pltpu.VMEM((16,), jnp.int32),
            pltpu.VMEM((CHUNK, D1), jnp.float32),
            pltpu.VMEM((CHUNK, D1), jnp.float32),
            pltpu.VMEM((ACC_R * D1,), jnp.float32),
            pltpu.SemaphoreType.DMA,
            pltpu.SemaphoreType.DMA,
            pltpu.SemaphoreType.DMA,
            pltpu.SemaphoreType.DMA,
        ],
    )
    def k(feat_hbm, lst_hbm, cnt_hbm, z_hbm, out_hbm,
          ilA, ilB, cnt_v, bufA, bufB, acc, isA, isB, gsA, gsB):
        c = lax.axis_index("c")
        s = lax.axis_index("s")
        wid = c * NS + s
        lbase = wid * 2 * CAP

        pltpu.sync_copy(z_hbm.at[pl.ds(0, ACC_R * D1)], acc)
        pltpu.sync_copy(cnt_hbm.at[pl.ds(wid * 16, 16)], cnt_v)
        nch = cnt_v[pl.ds(0, 16)][0]

        def stage(j, il, sem):
            pltpu.async_copy(lst_hbm.at[pl.ds(lbase + j * 2 * CHUNK, 2 * CHUNK)],
                             il, sem)

        def gather(il, buf, gsem):
            pltpu.async_copy(feat_hbm.at[il.at[pl.ds(0, CHUNK)]], buf, gsem)

        def wait_list(il, sem):
            pltpu.make_async_copy(lst_hbm.at[pl.ds(0, 2 * CHUNK)], il, sem).wait()

        def wait_rows(buf, gsem):
            pltpu.make_async_copy(feat_hbm.at[pl.ds(0, CHUNK)], buf, gsem).wait()

        def accumulate(il, buf):
            def grp(g, _):
                dlv = il[pl.ds(CHUNK + g * 16, 16)] * D1
                for k_ in range(16):
                    ob = dlv[k_]
                    e = g * 16 + k_
                    for i_ in range(D1 // 16):
                        plsc.addupdate(
                            acc.at[pl.ds(ob + i_ * 16, 16)],
                            buf[e, pl.ds(i_ * 16, 16)])
                return 0
            lax.fori_loop(0, CHUNK // 16, grp, 0)

        @pl.when(nch > 0)
        def _():
            stage(0, ilA, isA)
            wait_list(ilA, isA)
            gather(ilA, bufA, gsA)

        @pl.when(nch > 1)
        def _():
            stage(1, ilB, isB)

        npairs = (nch + 1) // 2

        def pair(p, _):
            j0 = p * 2
            j1 = j0 + 1
            @pl.when(j1 < nch)
            def _():
                wait_list(ilB, isB)
                gather(ilB, bufB, gsB)
            wait_rows(bufA, gsA)
            accumulate(ilA, bufA)
            @pl.when(j0 + 2 < nch)
            def _():
                stage(j0 + 2, ilA, isA)
            @pl.when(j1 < nch)
            def _():
                @pl.when(j1 + 1 < nch)
                def _():
                    wait_list(ilA, isA)
                    gather(ilA, bufA, gsA)
                wait_rows(bufB, gsB)
                accumulate(ilB, bufB)
                @pl.when(j1 + 2 < nch)
                def _():
                    stage(j1 + 2, ilB, isB)
            return 0
        lax.fori_loop(0, npairs, pair, 0)

        pltpu.sync_copy(acc.at[pl.ds(0, STRIPE * D1)],
                        out_hbm.at[pl.ds((c * HP + s * STRIPE) * D1, STRIPE * D1)])

    return k(featp, lst, cnt, zflat)


def _decode(s_arr, t_arr, pei):
    @functools.partial(
        pl.kernel,
        mesh=_sc_mesh(),
        compiler_params=pltpu.CompilerParams(needs_layout_passes=False),
        out_type=jax.ShapeDtypeStruct((P,), jnp.float32),
        scratch_types=[
            pltpu.VMEM((N,), jnp.float32),
            pltpu.VMEM((N,), jnp.float32),
            pltpu.VMEM((PC,), jnp.int32),
            pltpu.VMEM((PC,), jnp.int32),
            pltpu.VMEM((PC,), jnp.float32),
        ],
    )
    def k(s_hbm, t_hbm, pei_hbm, out_hbm, sv, tv, rv, cv, ov):
        c = lax.axis_index("c")
        s = lax.axis_index("s")
        wid = c * NS + s
        off = jnp.minimum(wid * PC, P - PC)

        pltpu.sync_copy(s_hbm, sv)
        pltpu.sync_copy(t_hbm, tv)
        pltpu.sync_copy(pei_hbm.at[pl.ds(off, PC)], rv)
        pltpu.sync_copy(pei_hbm.at[pl.ds(P + off, PC)], cv)

        def body(g, _):
            ir = rv[pl.ds(g * 16, 16)]
            ic = cv[pl.ds(g * 16, 16)]
            vs = plsc.load_gather(sv, [ir])
            vt = plsc.load_gather(tv, [ic])
            u = vs + vt
            ov[pl.ds(g * 16, 16)] = 1.0 / (1.0 + jnp.exp(-u))
            return 0
        lax.fori_loop(0, PC // 16, body, 0)

        pltpu.sync_copy(ov, out_hbm.at[pl.ds(off, PC)])

    return k(s_arr, t_arr, pei)


_BLK = 1000


def _tc_scale(deg, x):
    def body(deg_ref, x_ref, dinv_ref, fp_ref):
        dv = lax.rsqrt(deg_ref[...] + 1.0)
        dinv_ref[...] = dv
        fp_ref[...] = x_ref[...] * dv

    return pl.pallas_call(
        body,
        grid=(N // _BLK,),
        in_specs=[pl.BlockSpec((_BLK, 1), lambda i: (i, 0)),
                  pl.BlockSpec((_BLK, D1), lambda i: (i, 0))],
        out_specs=[pl.BlockSpec((_BLK, 1), lambda i: (i, 0)),
                   pl.BlockSpec((_BLK, D1), lambda i: (i, 0))],
        out_shape=[jax.ShapeDtypeStruct((N, 1), jnp.float32),
                   jax.ShapeDtypeStruct((N, D1), jnp.float32)],
    )(deg, x)


def _tc_mlp(dinv, agg1, featp1, W1, b1, W2):
    def body(dinv_ref, agg_ref, fp_ref, W1_ref, b1_ref, W2_ref, out_ref):
        a = dinv_ref[...] * (agg_ref[...] + fp_ref[...])
        h = jnp.dot(a, W1_ref[...], preferred_element_type=jnp.float32)
        h = jnp.maximum(h + b1_ref[...], 0.0)
        hw = jnp.dot(h, W2_ref[...], preferred_element_type=jnp.float32)
        out_ref[...] = dinv_ref[...] * hw

    return pl.pallas_call(
        body,
        grid=(N // _BLK,),
        in_specs=[pl.BlockSpec((_BLK, 1), lambda i: (i, 0)),
                  pl.BlockSpec((_BLK, D1), lambda i: (i, 0)),
                  pl.BlockSpec((_BLK, D1), lambda i: (i, 0)),
                  pl.BlockSpec((D1, DH), lambda i: (0, 0)),
                  pl.BlockSpec((1, DH), lambda i: (0, 0)),
                  pl.BlockSpec((DH, D1), lambda i: (0, 0))],
        out_specs=pl.BlockSpec((_BLK, D1), lambda i: (i, 0)),
        out_shape=jax.ShapeDtypeStruct((N, D1), jnp.float32),
    )(dinv, agg1, featp1, W1, b1, W2)


def _tc_final(dinv, agg2, featp2, b2, Wcat, bvec):
    def body(dinv_ref, agg_ref, fp_ref, b2_ref, Wc_ref, bv_ref, out_ref):
        z = dinv_ref[...] * (agg_ref[...] + fp_ref[...]) + b2_ref[...]
        out_ref[...] = jnp.dot(z, Wc_ref[...],
                               preferred_element_type=jnp.float32) + bv_ref[...]

    return pl.pallas_call(
        body,
        grid=(N // _BLK,),
        in_specs=[pl.BlockSpec((_BLK, 1), lambda i: (i, 0)),
                  pl.BlockSpec((_BLK, D1), lambda i: (i, 0)),
                  pl.BlockSpec((_BLK, D1), lambda i: (i, 0)),
                  pl.BlockSpec((1, D1), lambda i: (0, 0)),
                  pl.BlockSpec((D1, 128), lambda i: (0, 0)),
                  pl.BlockSpec((1, 128), lambda i: (0, 0))],
        out_specs=pl.BlockSpec((_BLK, 128), lambda i: (i, 0)),
        out_shape=jax.ShapeDtypeStruct((N, 128), jnp.float32),
    )(dinv, agg2, featp2, b2, Wcat, bvec)


def kernel(x, edge_index, predict_edge_index, W1, b1, W2, b2, Wout, bout):
    zflat = jnp.zeros((ACC_R * D1,), jnp.float32)

    deg16, lst, cnt = _prep(edge_index.reshape(2 * E))
    dd = deg16.reshape(NC * HP, 16)[:, 0]
    deg = jnp.concatenate([dd[:H], dd[HP:HP + H]]).reshape(N, 1)

    dinv, featp1 = _tc_scale(deg, x)

    a1 = _agg(featp1, lst, cnt, zflat).reshape(NC * HP, D1)
    agg1 = jnp.concatenate([a1[:H], a1[HP:HP + H]])
    featp2 = _tc_mlp(dinv, agg1, featp1, W1, b1.reshape(1, DH), W2)
    a2 = _agg(featp2, lst, cnt, zflat).reshape(NC * HP, D1)
    agg2 = jnp.concatenate([a2[:H], a2[HP:HP + H]])

    Wcat = jnp.concatenate([Wout[:D1], Wout[D1:]], axis=1)
    Wcat = jnp.pad(Wcat, ((0, 0), (0, 126)))
    bvec = jnp.concatenate([bout, jnp.zeros((127,), jnp.float32)]).reshape(1, 128)
    st = _tc_final(dinv, agg2, featp2, b2.reshape(1, D1), Wcat, bvec)

    s_arr = st[:, 0]
    t_arr = st[:, 1]
    return _decode(s_arr, t_arr, predict_edge_index.reshape(2 * P))


# speedup vs baseline: 4.2130x; 1.0947x over previous
import functools

import jax
import jax.numpy as jnp
from jax import lax
from jax.experimental import pallas as pl
from jax.experimental.pallas import tpu as pltpu
from jax.experimental.pallas import tpu_sc as plsc

N = 10000
E = 160000
P = 20000
D1 = 256
DH = 512

NC = 2
NS = 16
H = N // NC
HP = 5120
STRIPE = HP // NS
EP = E // NS
CHUNK = 64
NCH = 158
CAP = NCH * CHUNK
ACC_R = STRIPE + 1

PC = 640


def _sc_mesh():
    return plsc.VectorSubcoreMesh(core_axis_name="c", subcore_axis_name="s",
                                  num_cores=NC)


def _prep(edge_index):
    @functools.partial(
        pl.kernel,
        mesh=_sc_mesh(),
        compiler_params=pltpu.CompilerParams(needs_layout_passes=False),
        out_type=[
            jax.ShapeDtypeStruct((NC * HP * 16,), jnp.float32),
            jax.ShapeDtypeStruct((NC * NS * 2 * CAP,), jnp.int32),
            jax.ShapeDtypeStruct((NC * NS * 16,), jnp.int32),
        ],
        scratch_types=[
            pltpu.VMEM((EP,), jnp.int32),
            pltpu.VMEM((EP,), jnp.int32),
            pltpu.VMEM((EP,), jnp.int32),
            pltpu.VMEM((EP,), jnp.int32),
            pltpu.SemaphoreType.DMA,
            pltpu.SemaphoreType.DMA,
            pltpu.VMEM((CAP + 16,), jnp.int32),
            pltpu.VMEM((CAP + 16,), jnp.int32),
            pltpu.VMEM((2 * CAP,), jnp.int32),
            pltpu.VMEM((ACC_R * 16,), jnp.float32),
            pltpu.VMEM((16,), jnp.int32),
        ],
    )
    def k(ei_hbm, deg_hbm, lst_hbm, cnt_hbm,
          srcA, dstA, srcB, dstB, ssA, ssB, csrc_v, cdl_v, ilist_v, dacc, cnt_v):
        c = lax.axis_index("c")
        s = lax.axis_index("s")
        wid = c * NS + s
        low = c * H + s * STRIPE
        high = jnp.minimum(low + STRIPE, c * H + H)

        def fill(i, _):
            csrc_v[pl.ds(i * 16, 16)] = jnp.zeros((16,), jnp.int32)
            cdl_v[pl.ds(i * 16, 16)] = jnp.full((16,), STRIPE, jnp.int32)
            return 0
        lax.fori_loop(0, (CAP + 16) // 16, fill, 0)

        def zdeg(i, _):
            dacc[pl.ds(i * 16, 16)] = jnp.zeros((16,), jnp.float32)
            return 0
        lax.fori_loop(0, ACC_R, zdeg, 0)

        def sstage(t, sv_ref, dv_ref, sem):
            pltpu.async_copy(ei_hbm.at[pl.ds(t * EP, EP)], sv_ref, sem)
            pltpu.async_copy(ei_hbm.at[pl.ds(E + t * EP, EP)], dv_ref, sem)

        def swait(sv_ref, dv_ref, sem):
            pltpu.make_async_copy(ei_hbm.at[pl.ds(0, EP)], sv_ref, sem).wait()
            pltpu.make_async_copy(ei_hbm.at[pl.ds(0, EP)], dv_ref, sem).wait()

        sstage(0, srcA, dstA, ssA)
        sstage(1, srcB, dstB, ssB)
        HCAP = CAP // 2

        def spair(p, carry):
            t0 = p * 2
            swait(srcA, dstA, ssA)
            swait(srcB, dstB, ssB)

            def both(g, c):
                nA, nB = c
                svA = srcA[pl.ds(g * 16, 16)]
                dvA = dstA[pl.ds(g * 16, 16)]
                svB = srcB[pl.ds(g * 16, 16)]
                dvB = dstB[pl.ds(g * 16, 16)]
                mA = jnp.logical_and(dvA >= low, dvA < high)
                mB = jnp.logical_and(dvB >= low, dvB < high)
                miA = mA.astype(jnp.int32)
                miB = mB.astype(jnp.int32)
                incA = plsc.cumsum(miA)
                incB = plsc.cumsum(miB)
                posA = jnp.where(mA, nA + incA - miA, CAP)
                posA = jnp.where(posA >= HCAP, CAP, posA)
                posB = jnp.minimum(jnp.where(mB, HCAP + nB + incB - miB, CAP), CAP)
                plsc.store_scatter(csrc_v, [posA], svA)
                plsc.store_scatter(cdl_v, [posA], dvA - low)
                plsc.store_scatter(csrc_v, [posB], svB)
                plsc.store_scatter(cdl_v, [posB], dvB - low)
                nA = jnp.minimum(nA + plsc.all_reduce_population_count(mA)[0], HCAP)
                nB = jnp.minimum(nB + plsc.all_reduce_population_count(mB)[0], HCAP)
                return (nA, nB)
            carry = lax.fori_loop(0, EP // 16, both, carry)

            @pl.when(t0 + 2 < NS)
            def _():
                sstage(t0 + 2, srcA, dstA, ssA)

            @pl.when(t0 + 3 < NS)
            def _():
                sstage(t0 + 3, srcB, dstB, ssB)
            return carry
        nA, nB = lax.fori_loop(0, NS // 2, spair,
                               (jnp.int32(0), jnp.int32(0)))

        def merge(i, _):
            sv = csrc_v[pl.ds(HCAP + i * 16, 16)]
            dv = cdl_v[pl.ds(HCAP + i * 16, 16)]
            csrc_v[pl.ds(nA + i * 16, 16)] = sv
            cdl_v[pl.ds(nA + i * 16, 16)] = dv
            return 0
        lax.fori_loop(0, (nB + 15) // 16, merge, 0)
        n = nA + nB
        nch = jnp.minimum((n + CHUNK - 1) // CHUNK, NCH)

        ones = jnp.ones((16,), jnp.float32)

        def dcount(g, _):
            dlv = cdl_v[pl.ds(g * 16, 16)]
            for k_ in range(16):
                o = dlv[k_]
                plsc.addupdate(dacc.at[pl.ds(o * 16, 16)], ones)
            return 0
        lax.fori_loop(0, nch * (CHUNK // 16), dcount, 0)

        def inter(j, _):
            def cpg(g, _):
                ilist_v[pl.ds(j * 2 * CHUNK + g * 16, 16)] = \
                    csrc_v[pl.ds(j * CHUNK + g * 16, 16)]
                ilist_v[pl.ds(j * 2 * CHUNK + CHUNK + g * 16, 16)] = \
                    cdl_v[pl.ds(j * CHUNK + g * 16, 16)]
                return 0
            lax.fori_loop(0, CHUNK // 16, cpg, 0)
            return 0
        lax.fori_loop(0, nch, inter, 0)

        pltpu.sync_copy(dacc.at[pl.ds(0, STRIPE * 16)],
                        deg_hbm.at[pl.ds((c * HP + s * STRIPE) * 16, STRIPE * 16)])
        pltpu.sync_copy(ilist_v, lst_hbm.at[pl.ds(wid * 2 * CAP, 2 * CAP)])
        cnt_v[pl.ds(0, 16)] = jnp.full((16,), 1, jnp.int32) * nch
        pltpu.sync_copy(cnt_v, cnt_hbm.at[pl.ds(wid * 16, 16)])

    return k(edge_index)


def _agg(featp, lst, cnt, zflat):
    @functools.partial(
        pl.kernel,
        mesh=_sc_mesh(),
        compiler_params=pltpu.CompilerParams(needs_layout_passes=False),
        out_type=jax.ShapeDtypeStruct((NC * HP * D1,), jnp.float32),
        scratch_types=[
            pltpu.VMEM((2 * CHUNK,), jnp.int32),
            pltpu.VMEM((2 * CHUNK,), jnp.int32),
            pltpu.VMEM((16,), jnp.int32),
            pltpu.VMEM((CHUNK, D1), jnp.float32),
            pltpu.VMEM((CHUNK, D1), jnp.float32),
            pltpu.VMEM((ACC_R * D1,), jnp.float32),
            pltpu.SemaphoreType.DMA,
            pltpu.SemaphoreType.DMA,
            pltpu.SemaphoreType.DMA,
            pltpu.SemaphoreType.DMA,
        ],
    )
    def k(feat_hbm, lst_hbm, cnt_hbm, z_hbm, out_hbm,
          ilA, ilB, cnt_v, bufA, bufB, acc, isA, isB, gsA, gsB):
        c = lax.axis_index("c")
        s = lax.axis_index("s")
        wid = c * NS + s
        lbase = wid * 2 * CAP

        pltpu.sync_copy(z_hbm.at[pl.ds(0, ACC_R * D1)], acc)
        pltpu.sync_copy(cnt_hbm.at[pl.ds(wid * 16, 16)], cnt_v)
        nch = cnt_v[pl.ds(0, 16)][0]

        def stage(j, il, sem):
            pltpu.async_copy(lst_hbm.at[pl.ds(lbase + j * 2 * CHUNK, 2 * CHUNK)],
                             il, sem)

        def gather(il, buf, gsem):
            pltpu.async_copy(feat_hbm.at[il.at[pl.ds(0, CHUNK)]], buf, gsem)

        def wait_list(il, sem):
            pltpu.make_async_copy(lst_hbm.at[pl.ds(0, 2 * CHUNK)], il, sem).wait()

        def wait_rows(buf, gsem):
            pltpu.make_async_copy(feat_hbm.at[pl.ds(0, CHUNK)], buf, gsem).wait()

        def accumulate(il, buf):
            def grp(g, _):
                dlv = il[pl.ds(CHUNK + g * 16, 16)] * D1
                for k_ in range(16):
                    ob = dlv[k_]
                    e = g * 16 + k_
                    for i_ in range(D1 // 16):
                        plsc.addupdate(
                            acc.at[pl.ds(ob + i_ * 16, 16)],
                            buf[e, pl.ds(i_ * 16, 16)])
                return 0
            lax.fori_loop(0, CHUNK // 16, grp, 0)

        @pl.when(nch > 0)
        def _():
            stage(0, ilA, isA)
            wait_list(ilA, isA)
            gather(ilA, bufA, gsA)

        @pl.when(nch > 1)
        def _():
            stage(1, ilB, isB)

        npairs = (nch + 1) // 2

        def pair(p, _):
            j0 = p * 2
            j1 = j0 + 1
            @pl.when(j1 < nch)
            def _():
                wait_list(ilB, isB)
                gather(ilB, bufB, gsB)
            wait_rows(bufA, gsA)
            accumulate(ilA, bufA)
            @pl.when(j0 + 2 < nch)
            def _():
                stage(j0 + 2, ilA, isA)
            @pl.when(j1 < nch)
            def _():
                @pl.when(j1 + 1 < nch)
                def _():
                    wait_list(ilA, isA)
                    gather(ilA, bufA, gsA)
                wait_rows(bufB, gsB)
                accumulate(ilB, bufB)
                @pl.when(j1 + 2 < nch)
                def _():
                    stage(j1 + 2, ilB, isB)
            return 0
        lax.fori_loop(0, npairs, pair, 0)

        pltpu.sync_copy(acc.at[pl.ds(0, STRIPE * D1)],
                        out_hbm.at[pl.ds((c * HP + s * STRIPE) * D1, STRIPE * D1)])

    return k(featp, lst, cnt, zflat)


def _decode(s_arr, t_arr, pei):
    @functools.partial(
        pl.kernel,
        mesh=_sc_mesh(),
        compiler_params=pltpu.CompilerParams(needs_layout_passes=False),
        out_type=jax.ShapeDtypeStruct((P,), jnp.float32),
        scratch_types=[
            pltpu.VMEM((N,), jnp.float32),
            pltpu.VMEM((N,), jnp.float32),
            pltpu.VMEM((PC,), jnp.int32),
            pltpu.VMEM((PC,), jnp.int32),
            pltpu.VMEM((PC,), jnp.float32),
        ],
    )
    def k(s_hbm, t_hbm, pei_hbm, out_hbm, sv, tv, rv, cv, ov):
        c = lax.axis_index("c")
        s = lax.axis_index("s")
        wid = c * NS + s
        off = jnp.minimum(wid * PC, P - PC)

        pltpu.sync_copy(s_hbm, sv)
        pltpu.sync_copy(t_hbm, tv)
        pltpu.sync_copy(pei_hbm.at[pl.ds(off, PC)], rv)
        pltpu.sync_copy(pei_hbm.at[pl.ds(P + off, PC)], cv)

        def body(g, _):
            ir = rv[pl.ds(g * 16, 16)]
            ic = cv[pl.ds(g * 16, 16)]
            vs = plsc.load_gather(sv, [ir])
            vt = plsc.load_gather(tv, [ic])
            u = vs + vt
            ov[pl.ds(g * 16, 16)] = 1.0 / (1.0 + jnp.exp(-u))
            return 0
        lax.fori_loop(0, PC // 16, body, 0)

        pltpu.sync_copy(ov, out_hbm.at[pl.ds(off, PC)])

    return k(s_arr, t_arr, pei)


_BLK = 1000


def _tc_scale(deg, x):
    def body(deg_ref, x_ref, dinv_ref, fp_ref):
        dv = lax.rsqrt(deg_ref[...] + 1.0)
        dinv_ref[...] = dv
        fp_ref[...] = x_ref[...] * dv

    return pl.pallas_call(
        body,
        grid=(N // _BLK,),
        in_specs=[pl.BlockSpec((_BLK, 1), lambda i: (i, 0)),
                  pl.BlockSpec((_BLK, D1), lambda i: (i, 0))],
        out_specs=[pl.BlockSpec((_BLK, 1), lambda i: (i, 0)),
                   pl.BlockSpec((_BLK, D1), lambda i: (i, 0))],
        out_shape=[jax.ShapeDtypeStruct((N, 1), jnp.float32),
                   jax.ShapeDtypeStruct((N, D1), jnp.float32)],
    )(deg, x)


def _tc_mlp(dinv, agg1, featp1, W1, b1, W2):
    def body(dinv_ref, agg_ref, fp_ref, W1_ref, b1_ref, W2_ref, out_ref):
        a = dinv_ref[...] * (agg_ref[...] + fp_ref[...])
        h = jnp.dot(a, W1_ref[...], preferred_element_type=jnp.float32)
        h = jnp.maximum(h + b1_ref[...], 0.0)
        hw = jnp.dot(h, W2_ref[...], preferred_element_type=jnp.float32)
        out_ref[...] = dinv_ref[...] * hw

    return pl.pallas_call(
        body,
        grid=(N // _BLK,),
        in_specs=[pl.BlockSpec((_BLK, 1), lambda i: (i, 0)),
                  pl.BlockSpec((_BLK, D1), lambda i: (i, 0)),
                  pl.BlockSpec((_BLK, D1), lambda i: (i, 0)),
                  pl.BlockSpec((D1, DH), lambda i: (0, 0)),
                  pl.BlockSpec((1, DH), lambda i: (0, 0)),
                  pl.BlockSpec((DH, D1), lambda i: (0, 0))],
        out_specs=pl.BlockSpec((_BLK, D1), lambda i: (i, 0)),
        out_shape=jax.ShapeDtypeStruct((N, D1), jnp.float32),
    )(dinv, agg1, featp1, W1, b1, W2)


def _tc_final(dinv, agg2, featp2, b2, Wcat, bvec):
    def body(dinv_ref, agg_ref, fp_ref, b2_ref, Wc_ref, bv_ref, out_ref):
        z = dinv_ref[...] * (agg_ref[...] + fp_ref[...]) + b2_ref[...]
        out_ref[...] = jnp.dot(z, Wc_ref[...],
                               preferred_element_type=jnp.float32) + bv_ref[...]

    return pl.pallas_call(
        body,
        grid=(N // _BLK,),
        in_specs=[pl.BlockSpec((_BLK, 1), lambda i: (i, 0)),
                  pl.BlockSpec((_BLK, D1), lambda i: (i, 0)),
                  pl.BlockSpec((_BLK, D1), lambda i: (i, 0)),
                  pl.BlockSpec((1, D1), lambda i: (0, 0)),
                  pl.BlockSpec((D1, 128), lambda i: (0, 0)),
                  pl.BlockSpec((1, 128), lambda i: (0, 0))],
        out_specs=pl.BlockSpec((_BLK, 128), lambda i: (i, 0)),
        out_shape=jax.ShapeDtypeStruct((N, 128), jnp.float32),
    )(dinv, agg2, featp2, b2, Wcat, bvec)


def kernel(x, edge_index, predict_edge_index, W1, b1, W2, b2, Wout, bout):
    zflat = jnp.zeros((ACC_R * D1,), jnp.float32)

    deg16, lst, cnt = _prep(edge_index.reshape(2 * E))
    dd = deg16.reshape(NC * HP, 16)[:, 0]
    deg = jnp.concatenate([dd[:H], dd[HP:HP + H]]).reshape(N, 1)

    dinv, featp1 = _tc_scale(deg, x)

    a1 = _agg(featp1, lst, cnt, zflat).reshape(NC * HP, D1)
    agg1 = jnp.concatenate([a1[:H], a1[HP:HP + H]])
    featp2 = _tc_mlp(dinv, agg1, featp1, W1, b1.reshape(1, DH), W2)
    a2 = _agg(featp2, lst, cnt, zflat).reshape(NC * HP, D1)
    agg2 = jnp.concatenate([a2[:H], a2[HP:HP + H]])

    Wcat = jnp.concatenate([Wout[:D1], Wout[D1:]], axis=1)
    Wcat = jnp.pad(Wcat, ((0, 0), (0, 126)))
    bvec = jnp.concatenate([bout, jnp.zeros((127,), jnp.float32)]).reshape(1, 128)
    st = _tc_final(dinv, agg2, featp2, b2.reshape(1, D1), Wcat, bvec)

    s_arr = st[:, 0]
    t_arr = st[:, 1]
    return _decode(s_arr, t_arr, predict_edge_index.reshape(2 * P))
